# baseline (speedup 1.0000x reference)
import functools

import jax
import jax.numpy as jnp
from jax import lax
from jax.experimental import pallas as pl
from jax.experimental.pallas import tpu as pltpu
from jax.experimental.pallas import tpu_sc as plsc

D_ = 2048
B_ = 4 * 8192
NW_ = 32
BPW_ = B_ // NW_
CH_ = 16
NCHUNK_ = BPW_ // CH_
NBUF_ = 3
LAG_ = 1


def _make_gather():
    mesh = plsc.VectorSubcoreMesh(core_axis_name="c", subcore_axis_name="s")

    @functools.partial(
        pl.kernel,
        mesh=mesh,
        out_type=jax.ShapeDtypeStruct((B_, D_), jnp.float32),
        scratch_types=[
            pltpu.VMEM((BPW_,), jnp.int32),
            pltpu.VMEM((NBUF_, CH_, D_), jnp.float32),
            pltpu.SemaphoreType.DMA((NBUF_,)),
            pltpu.SemaphoreType.DMA((NBUF_,)),
        ],
    )
    def gather_kernel(idx_hbm, table_hbm, out_hbm, idx_v, rows_v, gsem, ssem):
        wid = lax.axis_index("s") * 2 + lax.axis_index("c")
        base = wid * BPW_
        pltpu.sync_copy(idx_hbm.at[pl.ds(base, BPW_)], idx_v)

        def gather_src(c):
            return table_hbm.at[idx_v.at[pl.ds(c * CH_, CH_)]]

        def out_dst(c):
            return out_hbm.at[pl.ds(base + c * CH_, CH_)]

        def issue_gather(c):
            pltpu.async_copy(gather_src(c), rows_v.at[c % NBUF_],
                             gsem.at[c % NBUF_])

        def wait_gather(c):
            pltpu.make_async_copy(gather_src(c), rows_v.at[c % NBUF_],
                                  gsem.at[c % NBUF_]).wait()

        def issue_store(c):
            pltpu.async_copy(rows_v.at[c % NBUF_], out_dst(c),
                             ssem.at[c % NBUF_])

        def wait_store(c):
            pltpu.make_async_copy(rows_v.at[c % NBUF_], out_dst(c),
                                  ssem.at[c % NBUF_]).wait()

        for c in range(LAG_):
            issue_gather(c)
        for c in range(NCHUNK_):
            wait_gather(c)
            issue_store(c)
            if c >= LAG_:
                wait_store(c - LAG_)
            if c + LAG_ < NCHUNK_:
                issue_gather(c + LAG_)
        for c in range(NCHUNK_ - LAG_, NCHUNK_):
            wait_store(c)

    return gather_kernel


_gather = _make_gather()


@jax.jit
def kernel(indices, table):
    flat_idx = jnp.reshape(indices, (B_,)).astype(jnp.int32)
    out = _gather(flat_idx, table)
    return jnp.reshape(out, (indices.shape[0], indices.shape[1], D_))


# speedup vs baseline: 1.0145x; 1.0145x over previous
import functools

import jax
import jax.numpy as jnp
from jax import lax
from jax.experimental import pallas as pl
from jax.experimental.pallas import tpu as pltpu
from jax.experimental.pallas import tpu_sc as plsc

D_ = 2048
B_ = 4 * 8192
NW_ = 32
BPW_ = B_ // NW_
CH_ = 8
NCHUNK_ = BPW_ // CH_
NBUF_ = 4
LAG_ = 2


def _make_gather():
    mesh = plsc.VectorSubcoreMesh(core_axis_name="c", subcore_axis_name="s")

    @functools.partial(
        pl.kernel,
        mesh=mesh,
        out_type=jax.ShapeDtypeStruct((B_, D_), jnp.float32),
        scratch_types=[
            pltpu.VMEM((BPW_,), jnp.int32),
            pltpu.VMEM((NBUF_, CH_, D_), jnp.float32),
            pltpu.SemaphoreType.DMA((NBUF_,)),
            pltpu.SemaphoreType.DMA((NBUF_,)),
        ],
    )
    def gather_kernel(idx_hbm, table_hbm, out_hbm, idx_v, rows_v, gsem, ssem):
        wid = lax.axis_index("s") * 2 + lax.axis_index("c")
        base = wid * BPW_
        pltpu.sync_copy(idx_hbm.at[pl.ds(base, BPW_)], idx_v)

        def gather_src(c):
            return table_hbm.at[idx_v.at[pl.ds(c * CH_, CH_)]]

        def out_dst(c):
            return out_hbm.at[pl.ds(base + c * CH_, CH_)]

        def issue_gather(c):
            pltpu.async_copy(gather_src(c), rows_v.at[c % NBUF_],
                             gsem.at[c % NBUF_])

        def wait_gather(c):
            pltpu.make_async_copy(gather_src(c), rows_v.at[c % NBUF_],
                                  gsem.at[c % NBUF_]).wait()

        def issue_store(c):
            pltpu.async_copy(rows_v.at[c % NBUF_], out_dst(c),
                             ssem.at[c % NBUF_])

        def wait_store(c):
            pltpu.make_async_copy(rows_v.at[c % NBUF_], out_dst(c),
                                  ssem.at[c % NBUF_]).wait()

        for c in range(LAG_):
            issue_gather(c)
        for c in range(NCHUNK_):
            wait_gather(c)
            issue_store(c)
            if c >= LAG_:
                wait_store(c - LAG_)
            if c + LAG_ < NCHUNK_:
                issue_gather(c + LAG_)
        for c in range(NCHUNK_ - LAG_, NCHUNK_):
            wait_store(c)

    return gather_kernel


_gather = _make_gather()


@jax.jit
def kernel(indices, table):
    flat_idx = jnp.reshape(indices, (B_,)).astype(jnp.int32)
    out = _gather(flat_idx, table)
    return jnp.reshape(out, (indices.shape[0], indices.shape[1], D_))


# speedup vs baseline: 1.0212x; 1.0066x over previous
import functools

import jax
import jax.numpy as jnp
from jax import lax
from jax.experimental import pallas as pl
from jax.experimental.pallas import tpu as pltpu
from jax.experimental.pallas import tpu_sc as plsc

D_ = 2048
B_ = 4 * 8192
NW_ = 32
BPW_ = B_ // NW_
CH_ = 4
NCHUNK_ = BPW_ // CH_
NBUF_ = 8
PRIME_ = NBUF_ - 1


def _make_gather():
    mesh = plsc.VectorSubcoreMesh(core_axis_name="c", subcore_axis_name="s")

    @functools.partial(
        pl.kernel,
        mesh=mesh,
        out_type=jax.ShapeDtypeStruct((B_, D_), jnp.float32),
        scratch_types=[
            pltpu.VMEM((NCHUNK_, CH_), jnp.int32),
            pltpu.VMEM((NBUF_, CH_, D_), jnp.float32),
            pltpu.SemaphoreType.DMA((NBUF_,)),
            pltpu.SemaphoreType.DMA((NBUF_,)),
        ],
    )
    def gather_kernel(idx_hbm, table_hbm, out_hbm, idx_v, rows_v, gsem, ssem):
        wid = lax.axis_index("s") * 2 + lax.axis_index("c")
        base = wid * BPW_
        pltpu.sync_copy(idx_hbm.at[wid], idx_v)

        def gather_src(c):
            return table_hbm.at[idx_v.at[c]]

        def out_dst(c):
            return out_hbm.at[pl.ds(base + c * CH_, CH_)]

        def issue_gather(c, b):
            pltpu.async_copy(gather_src(c), rows_v.at[b], gsem.at[b])

        def step(c, b, refill):
            pltpu.make_async_copy(gather_src(c), rows_v.at[b],
                                  gsem.at[b]).wait()
            pltpu.async_copy(rows_v.at[b], out_dst(c), ssem.at[b])
            pltpu.make_async_copy(rows_v.at[b], out_dst(c),
                                  ssem.at[b]).wait()
            if refill:
                issue_gather(c + PRIME_, (b + PRIME_) % NBUF_)

        for c in range(PRIME_):
            issue_gather(c, c)

        @pl.loop(0, NCHUNK_ - NBUF_, step=NBUF_)
        def _grp(c0):
            for j in range(NBUF_):
                step(c0 + j, j, True)

        for c in range(NCHUNK_ - NBUF_, NCHUNK_):
            step(c, c % NBUF_, c + PRIME_ < NCHUNK_)

    return gather_kernel


_gather = _make_gather()


@jax.jit
def kernel(indices, table):
    flat_idx = jnp.reshape(indices, (NW_, NCHUNK_, CH_)).astype(jnp.int32)
    out = _gather(flat_idx, table)
    return jnp.reshape(out, (indices.shape[0], indices.shape[1], D_))


# speedup vs baseline: 1.0400x; 1.0184x over previous
import functools

import jax
import jax.numpy as jnp
from jax import lax
from jax.experimental import pallas as pl
from jax.experimental.pallas import tpu as pltpu
from jax.experimental.pallas import tpu_sc as plsc

D_ = 2048
B_ = 4 * 8192
NW_ = 32
BPW_ = B_ // NW_
CH_ = 8
NCHUNK_ = BPW_ // CH_
NBUF_ = 4


def _make_gather():
    mesh = plsc.VectorSubcoreMesh(core_axis_name="c", subcore_axis_name="s")

    @functools.partial(
        pl.kernel,
        mesh=mesh,
        out_type=jax.ShapeDtypeStruct((B_, D_), jnp.float32),
        scratch_types=[
            pltpu.VMEM((NCHUNK_, CH_), jnp.int32),
            pltpu.VMEM((NBUF_, CH_, D_), jnp.float32),
            pltpu.SemaphoreType.DMA((NBUF_,)),
            pltpu.SemaphoreType.DMA((NBUF_,)),
        ],
    )
    def gather_kernel(idx_hbm, table_hbm, out_hbm, idx_v, rows_v, gsem, ssem):
        wid = lax.axis_index("s") * 2 + lax.axis_index("c")
        base = wid * BPW_
        pltpu.sync_copy(idx_hbm.at[wid], idx_v)

        def gather_src(c):
            return table_hbm.at[idx_v.at[c]]

        def out_dst(c):
            return out_hbm.at[pl.ds(base + c * CH_, CH_)]

        for b in range(NBUF_):
            pltpu.async_copy(gather_src(b), rows_v.at[b], gsem.at[b])

        def drain_and_store(c, b):
            pltpu.make_async_copy(gather_src(c), rows_v.at[b],
                                  gsem.at[b]).wait()
            pltpu.async_copy(rows_v.at[b], out_dst(c), ssem.at[b])
            pltpu.make_async_copy(rows_v.at[b], out_dst(c),
                                  ssem.at[b]).wait()

        @pl.loop(0, NCHUNK_ - NBUF_, step=NBUF_)
        def _grp(c0):
            for b in range(NBUF_):
                c = c0 + b
                drain_and_store(c, b)
                pltpu.async_copy(gather_src(c + NBUF_), rows_v.at[b],
                                 gsem.at[b])

        for b in range(NBUF_):
            drain_and_store(NCHUNK_ - NBUF_ + b, b)

    return gather_kernel


_gather = _make_gather()


@jax.jit
def kernel(indices, table):
    flat_idx = jnp.reshape(indices, (NW_, NCHUNK_, CH_)).astype(jnp.int32)
    out = _gather(flat_idx, table)
    return jnp.reshape(out, (indices.shape[0], indices.shape[1], D_))


# speedup vs baseline: 1.0501x; 1.0098x over previous
import functools

import jax
import jax.numpy as jnp
from jax import lax
from jax.experimental import pallas as pl
from jax.experimental.pallas import tpu as pltpu
from jax.experimental.pallas import tpu_sc as plsc

D_ = 2048
B_ = 4 * 8192
NW_ = 32
BPW_ = B_ // NW_
CH_ = 8
NCHUNK_ = BPW_ // CH_
NBUF_ = 4


def _make_gather():
    mesh = plsc.VectorSubcoreMesh(core_axis_name="c", subcore_axis_name="s")

    @functools.partial(
        pl.kernel,
        mesh=mesh,
        out_type=jax.ShapeDtypeStruct((B_, D_), jnp.float32),
        scratch_types=[
            pltpu.VMEM((BPW_,), jnp.int32),
            pltpu.VMEM((NBUF_, CH_, D_), jnp.float32),
            pltpu.SemaphoreType.DMA((NBUF_,)),
            pltpu.SemaphoreType.DMA((NBUF_,)),
        ],
    )
    def gather_kernel(idx_hbm, table_hbm, out_hbm, idx_v, rows_v, gsem, ssem):
        wid = lax.axis_index("s") * 2 + lax.axis_index("c")
        base = wid * BPW_
        pltpu.sync_copy(idx_hbm.at[pl.ds(base, BPW_)], idx_v)

        def gather_src(c):
            return table_hbm.at[idx_v.at[pl.ds(c * CH_, CH_)]]

        def out_dst(c):
            return out_hbm.at[pl.ds(base + c * CH_, CH_)]

        for b in range(NBUF_):
            pltpu.async_copy(gather_src(b), rows_v.at[b], gsem.at[b])

        def drain_and_store(c, b):
            pltpu.make_async_copy(gather_src(c), rows_v.at[b],
                                  gsem.at[b]).wait()
            pltpu.async_copy(rows_v.at[b], out_dst(c), ssem.at[b])
            pltpu.make_async_copy(rows_v.at[b], out_dst(c),
                                  ssem.at[b]).wait()

        @pl.loop(0, NCHUNK_ - NBUF_, step=NBUF_)
        def _grp(c0):
            for b in range(NBUF_):
                c = c0 + b
                drain_and_store(c, b)
                pltpu.async_copy(gather_src(c + NBUF_), rows_v.at[b],
                                 gsem.at[b])

        for b in range(NBUF_):
            drain_and_store(NCHUNK_ - NBUF_ + b, b)

    return gather_kernel


_gather = _make_gather()


@jax.jit
def kernel(indices, table):
    flat_idx = jnp.reshape(indices, (B_,)).astype(jnp.int32)
    out = _gather(flat_idx, table)
    return jnp.reshape(out, (indices.shape[0], indices.shape[1], D_))


# speedup vs baseline: 1.0557x; 1.0053x over previous
import functools

import jax
import jax.numpy as jnp
from jax import lax
from jax.experimental import pallas as pl
from jax.experimental.pallas import tpu as pltpu
from jax.experimental.pallas import tpu_sc as plsc

D_ = 2048
B_ = 4 * 8192
NW_ = 32
BPW_ = B_ // NW_
CH_ = 8
NCHUNK_ = BPW_ // CH_
NBUF_ = 7
_MAIN_ = ((NCHUNK_ - NBUF_) // NBUF_) * NBUF_


def _make_gather():
    mesh = plsc.VectorSubcoreMesh(core_axis_name="c", subcore_axis_name="s")

    @functools.partial(
        pl.kernel,
        mesh=mesh,
        out_type=jax.ShapeDtypeStruct((B_, D_), jnp.float32),
        scratch_types=[
            pltpu.VMEM((BPW_,), jnp.int32),
            pltpu.VMEM((NBUF_, CH_, D_), jnp.float32),
            pltpu.SemaphoreType.DMA((NBUF_,)),
            pltpu.SemaphoreType.DMA((NBUF_,)),
        ],
    )
    def gather_kernel(idx_hbm, table_hbm, out_hbm, idx_v, rows_v, gsem, ssem):
        wid = lax.axis_index("s") * 2 + lax.axis_index("c")
        base = wid * BPW_
        pltpu.sync_copy(idx_hbm.at[pl.ds(base, BPW_)], idx_v)

        def gather_src(c):
            return table_hbm.at[idx_v.at[pl.ds(c * CH_, CH_)]]

        def out_dst(c):
            return out_hbm.at[pl.ds(base + c * CH_, CH_)]

        def step(c, b, refill):
            pltpu.make_async_copy(gather_src(c), rows_v.at[b],
                                  gsem.at[b]).wait()
            pltpu.async_copy(rows_v.at[b], out_dst(c), ssem.at[b])
            pltpu.make_async_copy(rows_v.at[b], out_dst(c),
                                  ssem.at[b]).wait()
            if refill:
                pltpu.async_copy(gather_src(c + NBUF_), rows_v.at[b],
                                 gsem.at[b])

        for b in range(NBUF_):
            pltpu.async_copy(gather_src(b), rows_v.at[b], gsem.at[b])

        @pl.loop(0, _MAIN_, step=NBUF_)
        def _grp(c0):
            for j in range(NBUF_):
                step(c0 + j, j, True)

        for c in range(_MAIN_, NCHUNK_):
            step(c, c % NBUF_, c + NBUF_ < NCHUNK_)

    return gather_kernel


_gather = _make_gather()


@jax.jit
def kernel(indices, table):
    flat_idx = jnp.reshape(indices, (B_,)).astype(jnp.int32)
    out = _gather(flat_idx, table)
    return jnp.reshape(out, (indices.shape[0], indices.shape[1], D_))
